# baseline (device time: 37779 ns/iter reference)
import jax
import jax.numpy as jnp
from jax import lax
from jax.experimental import pallas as pl
from jax.experimental.pallas import tpu as pltpu

N_DEV = 16
B = 256
D = 256
H = 512
BLK = B // N_DEV
G = 4
NG = N_DEV // G

CDT = jnp.bfloat16


def _antipode(i: int) -> int:
    z, k = i // 4, i % 4
    return 4 * (3 - z) + ((k + 2) % 4)


def kernel(x, Win0, Wout0, Win1, Wout1, Win2, Wout2):
    def body(
        x_hbm,
        win0_hbm,
        wout0_hbm,
        win1_hbm,
        wout1_hbm,
        win2_hbm,
        wout2_hbm,
        out_hbm,
        x_ref,
        win_ref,
        wout_ref,
        out_stage,
        p_ref,
        r_ref,
        rs_buf,
        ag_buf,
        rs_sems,
        ag_sems,
        rs_send_sems,
        ag_send_sems,
        ldma_sems,
    ):
        my_id = lax.axis_index("i")

        in_copies = [
            pltpu.make_async_copy(x_hbm, x_ref, ldma_sems.at[0]),
            pltpu.make_async_copy(win0_hbm, win_ref.at[0], ldma_sems.at[1]),
            pltpu.make_async_copy(wout0_hbm, wout_ref.at[0], ldma_sems.at[2]),
            pltpu.make_async_copy(win1_hbm, win_ref.at[1], ldma_sems.at[3]),
            pltpu.make_async_copy(wout1_hbm, wout_ref.at[1], ldma_sems.at[4]),
            pltpu.make_async_copy(win2_hbm, win_ref.at[2], ldma_sems.at[5]),
            pltpu.make_async_copy(wout2_hbm, wout_ref.at[2], ldma_sems.at[6]),
        ]
        for c in in_copies:
            c.start()

        def rs_send_desc(b, dest):
            return pltpu.make_async_remote_copy(
                src_ref=p_ref.at[pl.ds(b * BLK, BLK), :],
                dst_ref=rs_buf.at[my_id],
                send_sem=rs_send_sems.at[dest],
                recv_sem=rs_sems.at[my_id],
                device_id=(dest,),
                device_id_type=pl.DeviceIdType.MESH,
            )

        def rs_recv_desc(s):
            return pltpu.make_async_remote_copy(
                src_ref=p_ref.at[pl.ds(0, BLK), :],
                dst_ref=rs_buf.at[s],
                send_sem=rs_send_sems.at[s],
                recv_sem=rs_sems.at[s],
                device_id=(s,),
                device_id_type=pl.DeviceIdType.MESH,
            )

        def ag_send_desc(j):
            return pltpu.make_async_remote_copy(
                src_ref=r_ref,
                dst_ref=ag_buf.at[my_id],
                send_sem=ag_send_sems.at[j],
                recv_sem=ag_sems.at[my_id],
                device_id=(j,),
                device_id_type=pl.DeviceIdType.MESH,
            )

        def ag_recv_desc(s):
            return pltpu.make_async_remote_copy(
                src_ref=r_ref,
                dst_ref=ag_buf.at[s],
                send_sem=ag_send_sems.at[s],
                recv_sem=ag_sems.at[s],
                device_id=(s,),
                device_id_type=pl.DeviceIdType.MESH,
            )

        def rs_start_block(b, dest):

            @pl.when(my_id != dest)
            def _():
                rs_send_desc(b, dest).start()

            @pl.when(my_id == dest)
            def _():
                rs_buf[dest, :, :] = p_ref[pl.ds(b * BLK, BLK), :]

        def rs_finish():
            R = jnp.zeros((BLK, D), jnp.float32)
            for s in range(N_DEV):

                @pl.when(my_id != s)
                def _(s=s):
                    rs_recv_desc(s).wait_recv()

                R = R + rs_buf[s, :, :].astype(jnp.float32)
            for j in range(N_DEV):

                @pl.when(my_id != j)
                def _(j=j):
                    rs_send_desc(0, j).wait_send()

            return R

        def ag_start(R, drain_prev):
            if drain_prev:
                for j in range(N_DEV):

                    @pl.when(my_id != j)
                    def _(j=j):
                        ag_send_desc(j).wait_send()

            r_ref[:, :] = R.astype(CDT)
            for j in range(N_DEV):

                @pl.when(my_id != j)
                def _(j=j):
                    ag_send_desc(j).start()

                @pl.when(my_id == j)
                def _(j=j):
                    ag_buf[j, :, :] = r_ref[:, :]

        def mlp_chunk(xv, win, wout):
            h = jnp.maximum(
                jnp.dot(xv, win, preferred_element_type=jnp.float32), 0.0
            )
            return jnp.dot(
                h.astype(CDT), wout, preferred_element_type=jnp.float32
            )

        def pipelined_layer(l, slot_of_block, dest_of_block):
            in_copies[2 * l + 1].wait()
            in_copies[2 * l + 2].wait()
            win = win_ref[l, :, :].astype(CDT)
            wout = wout_ref[l, :, :].astype(CDT)
            for g in range(NG):
                blocks = list(range(g * G, (g + 1) * G))
                for b in blocks:
                    s = slot_of_block(b)

                    @pl.when(my_id != s)
                    def _(s=s):
                        ag_recv_desc(s).wait_recv()

                xg = jnp.concatenate(
                    [ag_buf[slot_of_block(b), :, :] for b in blocks], axis=0
                )
                pg = mlp_chunk(xg, win, wout)
                p_ref[pl.ds(g * G * BLK, G * BLK), :] = pg.astype(CDT)
                for b in blocks:
                    rs_start_block(b, dest_of_block(b))

        in_copies[0].wait()
        in_copies[1].wait()
        in_copies[2].wait()
        p_ref[:, :] = mlp_chunk(
            x_ref[:, :].astype(CDT),
            win_ref[0, :, :].astype(CDT),
            wout_ref[0, :, :].astype(CDT),
        ).astype(CDT)
        for b in range(N_DEV):
            rs_start_block(b, b)
        R = rs_finish()

        ag_start(R, drain_prev=False)
        pipelined_layer(1, slot_of_block=lambda b: b, dest_of_block=_antipode)
        R = rs_finish()

        ag_start(R, drain_prev=True)
        pipelined_layer(2, slot_of_block=_antipode, dest_of_block=lambda b: b)
        R = rs_finish()

        out_stage[:, :] = R
        out_copy = pltpu.make_async_copy(out_stage, out_hbm, ldma_sems.at[7])
        out_copy.start()
        out_copy.wait()

        for j in range(N_DEV):

            @pl.when(my_id != j)
            def _(j=j):
                ag_send_desc(j).wait_send()

    return pl.pallas_call(
        body,
        out_shape=jax.ShapeDtypeStruct((BLK, D), jnp.float32),
        in_specs=[pl.BlockSpec(memory_space=pltpu.MemorySpace.HBM)] * 7,
        out_specs=pl.BlockSpec(memory_space=pltpu.MemorySpace.HBM),
        scratch_shapes=[
            pltpu.VMEM((B, D), jnp.float32),
            pltpu.VMEM((3, B, H), jnp.float32),
            pltpu.VMEM((3, H, D), jnp.float32),
            pltpu.VMEM((BLK, D), jnp.float32),
            pltpu.VMEM((B, D), CDT),
            pltpu.VMEM((BLK, D), CDT),
            pltpu.VMEM((N_DEV, BLK, D), CDT),
            pltpu.VMEM((N_DEV, BLK, D), CDT),
            pltpu.SemaphoreType.DMA((N_DEV,)),
            pltpu.SemaphoreType.DMA((N_DEV,)),
            pltpu.SemaphoreType.DMA((N_DEV,)),
            pltpu.SemaphoreType.DMA((N_DEV,)),
            pltpu.SemaphoreType.DMA((8,)),
        ],
    )(x, Win0, Wout0, Win1, Wout1, Win2, Wout2)


# device time: 31721 ns/iter; 1.1910x vs baseline; 1.1910x over previous
import jax
import jax.numpy as jnp
from jax import lax
from jax.experimental import pallas as pl
from jax.experimental.pallas import tpu as pltpu

N_DEV = 16
B = 256
D = 256
H = 512
BLK = B // N_DEV
G = 4
NG = N_DEV // G

CDT = jnp.bfloat16


def _antipode(i: int) -> int:
    z, k = i // 4, i % 4
    return 4 * (3 - z) + ((k + 2) % 4)


def kernel(x, Win0, Wout0, Win1, Wout1, Win2, Wout2):
    def body(
        x_hbm,
        win0_hbm,
        wout0_hbm,
        win1_hbm,
        wout1_hbm,
        win2_hbm,
        wout2_hbm,
        out_hbm,
        x_ref,
        win_ref,
        wout_ref,
        out_stage,
        p_ref,
        r_ref,
        rs_buf,
        ag_buf,
        rs_sems,
        ag_sems,
        rs_send_sems,
        ag_send_sems,
        ldma_sems,
    ):
        my_id = lax.axis_index("i")

        in_copies = [
            pltpu.make_async_copy(x_hbm, x_ref, ldma_sems.at[0]),
            pltpu.make_async_copy(win0_hbm, win_ref.at[0], ldma_sems.at[1]),
            pltpu.make_async_copy(wout0_hbm, wout_ref.at[0], ldma_sems.at[2]),
            pltpu.make_async_copy(win1_hbm, win_ref.at[1], ldma_sems.at[3]),
            pltpu.make_async_copy(wout1_hbm, wout_ref.at[1], ldma_sems.at[4]),
            pltpu.make_async_copy(win2_hbm, win_ref.at[2], ldma_sems.at[5]),
            pltpu.make_async_copy(wout2_hbm, wout_ref.at[2], ldma_sems.at[6]),
        ]
        for c in in_copies:
            c.start()

        def rs_send_desc(b, dest):
            return pltpu.make_async_remote_copy(
                src_ref=p_ref.at[pl.ds(b * BLK, BLK), :],
                dst_ref=rs_buf.at[my_id],
                send_sem=rs_send_sems.at[dest],
                recv_sem=rs_sems.at[my_id],
                device_id=(dest,),
                device_id_type=pl.DeviceIdType.MESH,
            )

        def rs_recv_desc(s):
            return pltpu.make_async_remote_copy(
                src_ref=p_ref.at[pl.ds(0, BLK), :],
                dst_ref=rs_buf.at[s],
                send_sem=rs_send_sems.at[s],
                recv_sem=rs_sems.at[s],
                device_id=(s,),
                device_id_type=pl.DeviceIdType.MESH,
            )

        def ag_send_desc(j):
            return pltpu.make_async_remote_copy(
                src_ref=r_ref,
                dst_ref=ag_buf.at[my_id],
                send_sem=ag_send_sems.at[j],
                recv_sem=ag_sems.at[my_id],
                device_id=(j,),
                device_id_type=pl.DeviceIdType.MESH,
            )

        def ag_recv_desc(s):
            return pltpu.make_async_remote_copy(
                src_ref=r_ref,
                dst_ref=ag_buf.at[s],
                send_sem=ag_send_sems.at[s],
                recv_sem=ag_sems.at[s],
                device_id=(s,),
                device_id_type=pl.DeviceIdType.MESH,
            )

        def rs_start_block(b, dest):

            @pl.when(my_id != dest)
            def _():
                rs_send_desc(b, dest).start()

            @pl.when(my_id == dest)
            def _():
                rs_buf[dest, :, :] = p_ref[pl.ds(b * BLK, BLK), :]

        def rs_finish():
            R = jnp.zeros((BLK, D), jnp.float32)
            for s in range(N_DEV):

                @pl.when(my_id != s)
                def _(s=s):
                    rs_recv_desc(s).wait_recv()

                R = R + rs_buf[s, :, :].astype(jnp.float32)
            for j in range(N_DEV):

                @pl.when(my_id != j)
                def _(j=j):
                    rs_send_desc(0, j).wait_send()

            return R

        def ag_start(R, drain_prev):
            if drain_prev:
                for j in range(N_DEV):

                    @pl.when(my_id != j)
                    def _(j=j):
                        ag_send_desc(j).wait_send()

            r_ref[:, :] = R.astype(CDT)
            for j in range(N_DEV):

                @pl.when(my_id != j)
                def _(j=j):
                    ag_send_desc(j).start()

                @pl.when(my_id == j)
                def _(j=j):
                    ag_buf[j, :, :] = r_ref[:, :]

        def mlp_chunk(xv, win, wout):
            h = jnp.maximum(
                jnp.dot(xv, win, preferred_element_type=jnp.float32), 0.0
            )
            return jnp.dot(
                h.astype(CDT), wout, preferred_element_type=jnp.float32
            )

        def pipelined_layer(l, slot_of_block, dest_of_block):
            in_copies[2 * l + 1].wait()
            in_copies[2 * l + 2].wait()
            win = win_ref[l, :, :].astype(CDT)
            wout = wout_ref[l, :, :].astype(CDT)
            for g in range(NG):
                blocks = list(range(g * G, (g + 1) * G))
                for b in blocks:
                    s = slot_of_block(b)

                    @pl.when(my_id != s)
                    def _(s=s):
                        ag_recv_desc(s).wait_recv()

                xg = jnp.concatenate(
                    [ag_buf[slot_of_block(b), :, :] for b in blocks], axis=0
                )
                pg = mlp_chunk(xg, win, wout)
                p_ref[pl.ds(g * G * BLK, G * BLK), :] = pg.astype(CDT)
                for b in blocks:
                    rs_start_block(b, dest_of_block(b))

        in_copies[0].wait()
        in_copies[1].wait()
        in_copies[2].wait()
        p_ref[:, :] = mlp_chunk(
            x_ref[:, :].astype(CDT),
            win_ref[0, :, :].astype(CDT),
            wout_ref[0, :, :].astype(CDT),
        ).astype(CDT)
        for b in range(N_DEV):
            rs_start_block(b, b)
        R = rs_finish()

        ag_start(R, drain_prev=False)
        pipelined_layer(1, slot_of_block=lambda b: b, dest_of_block=_antipode)
        R = rs_finish()

        ag_start(R, drain_prev=True)
        pipelined_layer(2, slot_of_block=_antipode, dest_of_block=lambda b: b)
        R = rs_finish()

        out_stage[:, :] = R
        out_copy = pltpu.make_async_copy(out_stage, out_hbm, ldma_sems.at[7])
        out_copy.start()
        out_copy.wait()

        for j in range(N_DEV):

            @pl.when(my_id != j)
            def _(j=j):
                ag_send_desc(j).wait_send()

    hbm = lambda a: pltpu.with_memory_space_constraint(
        a, pltpu.MemorySpace.HBM
    )
    return pl.pallas_call(
        body,
        out_shape=jax.ShapeDtypeStruct((BLK, D), jnp.float32),
        in_specs=[pl.BlockSpec(memory_space=pltpu.MemorySpace.HBM)] * 7,
        out_specs=pl.BlockSpec(memory_space=pltpu.MemorySpace.HBM),
        scratch_shapes=[
            pltpu.VMEM((B, D), jnp.float32),
            pltpu.VMEM((3, B, H), jnp.float32),
            pltpu.VMEM((3, H, D), jnp.float32),
            pltpu.VMEM((BLK, D), jnp.float32),
            pltpu.VMEM((B, D), CDT),
            pltpu.VMEM((BLK, D), CDT),
            pltpu.VMEM((N_DEV, BLK, D), CDT),
            pltpu.VMEM((N_DEV, BLK, D), CDT),
            pltpu.SemaphoreType.DMA((N_DEV,)),
            pltpu.SemaphoreType.DMA((N_DEV,)),
            pltpu.SemaphoreType.DMA((N_DEV,)),
            pltpu.SemaphoreType.DMA((N_DEV,)),
            pltpu.SemaphoreType.DMA((8,)),
        ],
    )(hbm(x), hbm(Win0), hbm(Wout0), hbm(Win1), hbm(Wout1), hbm(Win2), hbm(Wout2))


# device time: 25169 ns/iter; 1.5010x vs baseline; 1.2603x over previous
import jax
import jax.numpy as jnp
from jax import lax
from jax.experimental import pallas as pl
from jax.experimental.pallas import tpu as pltpu

N_DEV = 16
B = 256
D = 256
H = 512
BLK = B // N_DEV
G = 4
NG = N_DEV // G

CDT = jnp.bfloat16


def _antipode(i: int) -> int:
    z, k = i // 4, i % 4
    return 4 * (3 - z) + ((k + 2) % 4)


def kernel(x, Win0, Wout0, Win1, Wout1, Win2, Wout2):
    def body(
        x_hbm,
        win0_hbm,
        wout0_hbm,
        win1_hbm,
        wout1_hbm,
        win2_hbm,
        wout2_hbm,
        out_hbm,
        x_ref,
        win_ref,
        wout_ref,
        out_stage,
        p_ref,
        r_ref,
        rs_buf,
        ag_buf,
        rs_sems,
        ag_sems,
        rs_send_sems,
        ag_send_sems,
        ldma_sems,
    ):
        my_id = lax.axis_index("i")

        in_copies = [
            pltpu.make_async_copy(x_hbm, x_ref, ldma_sems.at[0]),
            pltpu.make_async_copy(win0_hbm, win_ref.at[0], ldma_sems.at[1]),
            pltpu.make_async_copy(wout0_hbm, wout_ref.at[0], ldma_sems.at[2]),
            pltpu.make_async_copy(win1_hbm, win_ref.at[1], ldma_sems.at[3]),
            pltpu.make_async_copy(wout1_hbm, wout_ref.at[1], ldma_sems.at[4]),
            pltpu.make_async_copy(win2_hbm, win_ref.at[2], ldma_sems.at[5]),
            pltpu.make_async_copy(wout2_hbm, wout_ref.at[2], ldma_sems.at[6]),
        ]
        for c in in_copies:
            c.start()

        barrier_sem = pltpu.get_barrier_semaphore()
        for p in range(N_DEV):

            @pl.when(my_id != p)
            def _(p=p):
                pl.semaphore_signal(
                    barrier_sem,
                    inc=1,
                    device_id=(p,),
                    device_id_type=pl.DeviceIdType.MESH,
                )

        pl.semaphore_wait(barrier_sem, N_DEV - 1)

        def rs_send_desc(b, dest):
            return pltpu.make_async_remote_copy(
                src_ref=p_ref.at[pl.ds(b * BLK, BLK), :],
                dst_ref=rs_buf.at[my_id],
                send_sem=rs_send_sems.at[dest],
                recv_sem=rs_sems.at[my_id],
                device_id=(dest,),
                device_id_type=pl.DeviceIdType.MESH,
            )

        def rs_recv_desc(s):
            return pltpu.make_async_remote_copy(
                src_ref=p_ref.at[pl.ds(0, BLK), :],
                dst_ref=rs_buf.at[s],
                send_sem=rs_send_sems.at[s],
                recv_sem=rs_sems.at[s],
                device_id=(s,),
                device_id_type=pl.DeviceIdType.MESH,
            )

        def ag_send_desc(j):
            return pltpu.make_async_remote_copy(
                src_ref=r_ref,
                dst_ref=ag_buf.at[my_id],
                send_sem=ag_send_sems.at[j],
                recv_sem=ag_sems.at[my_id],
                device_id=(j,),
                device_id_type=pl.DeviceIdType.MESH,
            )

        def ag_recv_desc(s):
            return pltpu.make_async_remote_copy(
                src_ref=r_ref,
                dst_ref=ag_buf.at[s],
                send_sem=ag_send_sems.at[s],
                recv_sem=ag_sems.at[s],
                device_id=(s,),
                device_id_type=pl.DeviceIdType.MESH,
            )

        def rs_start_block(b, dest):

            @pl.when(my_id != dest)
            def _():
                rs_send_desc(b, dest).start()

            @pl.when(my_id == dest)
            def _():
                rs_buf[dest, :, :] = p_ref[pl.ds(b * BLK, BLK), :]

        def rs_finish():
            R = None
            for s in range(N_DEV):

                @pl.when(my_id != s)
                def _(s=s):
                    rs_recv_desc(s).wait_recv()

                v = rs_buf[s, :, :].astype(jnp.float32)
                R = v if R is None else R + v
            for j in range(N_DEV):

                @pl.when(my_id != j)
                def _(j=j):
                    rs_send_desc(0, j).wait_send()

            return R

        def ag_start(R, drain_prev):
            if drain_prev:
                for j in range(N_DEV):

                    @pl.when(my_id != j)
                    def _(j=j):
                        ag_send_desc(j).wait_send()

            r_ref[:, :] = R.astype(CDT)
            for j in range(N_DEV):

                @pl.when(my_id != j)
                def _(j=j):
                    ag_send_desc(j).start()

                @pl.when(my_id == j)
                def _(j=j):
                    ag_buf[j, :, :] = r_ref[:, :]

        def mlp_chunk(xv, win, wout):
            h = jnp.maximum(
                jnp.dot(xv, win, preferred_element_type=jnp.float32), 0.0
            )
            return jnp.dot(
                h.astype(CDT), wout, preferred_element_type=jnp.float32
            )

        def pipelined_layer(l, slot_of_block, dest_of_block):
            in_copies[2 * l + 1].wait()
            in_copies[2 * l + 2].wait()
            win = win_ref[l, :, :].astype(CDT)
            wout = wout_ref[l, :, :].astype(CDT)
            for g in range(NG):
                blocks = list(range(g * G, (g + 1) * G))
                for b in blocks:
                    s = slot_of_block(b)

                    @pl.when(my_id != s)
                    def _(s=s):
                        ag_recv_desc(s).wait_recv()

                xg = jnp.concatenate(
                    [ag_buf[slot_of_block(b), :, :] for b in blocks], axis=0
                )
                pg = mlp_chunk(xg, win, wout)
                p_ref[pl.ds(g * G * BLK, G * BLK), :] = pg.astype(CDT)
                for b in blocks:
                    rs_start_block(b, dest_of_block(b))

        in_copies[0].wait()
        in_copies[1].wait()
        in_copies[2].wait()
        p_ref[:, :] = mlp_chunk(
            x_ref[:, :].astype(CDT),
            win_ref[0, :, :].astype(CDT),
            wout_ref[0, :, :].astype(CDT),
        ).astype(CDT)
        for b in range(N_DEV):
            rs_start_block(b, b)
        R = rs_finish()

        ag_start(R, drain_prev=False)
        pipelined_layer(1, slot_of_block=lambda b: b, dest_of_block=_antipode)
        R = rs_finish()

        ag_start(R, drain_prev=True)
        pipelined_layer(2, slot_of_block=_antipode, dest_of_block=lambda b: b)
        R = rs_finish()

        out_stage[:, :] = R
        out_copy = pltpu.make_async_copy(out_stage, out_hbm, ldma_sems.at[7])
        out_copy.start()
        out_copy.wait()

        for j in range(N_DEV):

            @pl.when(my_id != j)
            def _(j=j):
                ag_send_desc(j).wait_send()

    hbm = lambda a: pltpu.with_memory_space_constraint(
        a, pltpu.MemorySpace.HBM
    )
    return pl.pallas_call(
        body,
        out_shape=jax.ShapeDtypeStruct((BLK, D), jnp.float32),
        in_specs=[pl.BlockSpec(memory_space=pltpu.MemorySpace.HBM)] * 7,
        out_specs=pl.BlockSpec(memory_space=pltpu.MemorySpace.HBM),
        compiler_params=pltpu.CompilerParams(collective_id=0),
        scratch_shapes=[
            pltpu.VMEM((B, D), jnp.float32),
            pltpu.VMEM((3, B, H), jnp.float32),
            pltpu.VMEM((3, H, D), jnp.float32),
            pltpu.VMEM((BLK, D), jnp.float32),
            pltpu.VMEM((B, D), CDT),
            pltpu.VMEM((BLK, D), CDT),
            pltpu.VMEM((N_DEV, BLK, D), CDT),
            pltpu.VMEM((N_DEV, BLK, D), CDT),
            pltpu.SemaphoreType.DMA((N_DEV,)),
            pltpu.SemaphoreType.DMA((N_DEV,)),
            pltpu.SemaphoreType.DMA((N_DEV,)),
            pltpu.SemaphoreType.DMA((N_DEV,)),
            pltpu.SemaphoreType.DMA((8,)),
        ],
    )(hbm(x), hbm(Win0), hbm(Wout0), hbm(Win1), hbm(Wout1), hbm(Win2), hbm(Wout2))
